# baseline (speedup 1.0000x reference)
import functools

import jax
import jax.numpy as jnp
from jax import lax
from jax.experimental import pallas as pl
from jax.experimental.pallas import tpu as pltpu
from jax.experimental.pallas import tpu_sc as plsc

L = 16
NC = 2
NS = 16
NW = NC * NS

_CES = [(0, 1), (2, 3), (4, 5), (6, 7),
        (0, 2), (1, 3), (4, 6), (5, 7),
        (1, 2), (5, 6),
        (0, 4), (1, 5), (2, 6), (3, 7),
        (2, 4), (3, 5),
        (1, 2), (3, 4), (5, 6)]


def _make_kernel(B, I, A, T, D, BC):
    NF = I // NW
    NG = BC // L
    NCH = B // BC
    HD = D // 2
    TS = HD + 1
    XS = NF * A + 1
    OS = NF * D + 1
    NTC = 2
    mesh = plsc.VectorSubcoreMesh(core_axis_name="c", subcore_axis_name="s",
                                  num_cores=NC, num_subcores=NS)

    @functools.partial(
        pl.kernel,
        out_type=jax.ShapeDtypeStruct((B, I * D), jnp.float32),
        mesh=mesh,
        scratch_types=[
            pltpu.VMEM((NTC * T * HD,), jnp.int32),
            pltpu.VMEM((NF * T * TS,), jnp.int32),
            pltpu.VMEM((BC, XS), jnp.float32),
            pltpu.VMEM((BC, OS), jnp.float32),
        ],
        compiler_params=pltpu.CompilerParams(
            needs_layout_passes=False, use_tc_tiling_on_sc=False),
    )
    def k(x2_hbm, pflat_hbm, y2_hbm, traw, tb, xb, ob):
        wid = lax.axis_index("s") * NC + lax.axis_index("c")
        f0 = wid * NF
        lane = jnp.arange(L, dtype=jnp.int32)
        pay = [jnp.full((L,), 1 << j, jnp.int32) for j in range(A)]
        j0_off = (255 ^ 15) * TS

        for tc in range(NF // NTC):
            pltpu.sync_copy(
                pflat_hbm.at[pl.ds((f0 + tc * NTC) * (T * HD), NTC * T * HD)],
                traw)

            @plsc.parallel_loop(0, NTC * T // 2, 1, unroll=4)
            def _srow(r2):
                w = traw[pl.ds(r2 * (2 * HD), L)]
                ri = r2 * 2 + (lane >> 3)
                m = ri & (T - 1)
                fc = ri >> 8
                s = m ^ (m >> 4)
                dst = ((tc * NTC + fc) * T + s) * TS + (lane & 7)
                plsc.store_scatter(tb, [dst], w)

        def cbody(ci, carry):
            b0c = ci * BC
            pltpu.sync_copy(
                x2_hbm.at[pl.ds(b0c, BC), pl.ds(f0 * A, NF * A)],
                xb.at[:, pl.ds(0, NF * A)])

            @plsc.parallel_loop(0, NF * NG, 1, unroll=2)
            def _group(t):
                fl = t // NG
                g = t - fl * NG
                tbase = fl * (T * TS)
                bloc = g * L + lane
                xrow = bloc * XS + fl * A
                v = [plsc.load_gather(xb, [bloc, jnp.full(
                        (L,), fl * A + j, jnp.int32)]) for j in range(A)]
                p = list(pay)
                for a, b in _CES:
                    c = v[a] <= v[b]
                    va, vb = v[a], v[b]
                    v[a] = jnp.where(c, va, vb)
                    v[b] = jnp.where(c, vb, va)
                    pa, pb = p[a], p[b]
                    p[a] = jnp.where(c, pa, pb)
                    p[b] = jnp.where(c, pb, pa)
                coef = [v[0]] + [v[j] - v[j - 1] for j in range(1, A)]
                m = [None] * A
                m[A - 1] = p[A - 1]
                for j in range(A - 2, 0, -1):
                    m[j] = m[j + 1] + p[j]
                base = [None] * A
                base[0] = jnp.full((L,), 0, jnp.int32) + (tbase + j0_off)
                for j in range(1, A):
                    s = m[j] ^ (m[j] >> 4)
                    base[j] = tbase + s * TS
                himask = jnp.full((L,), -65536, jnp.int32)
                for dp in range(HD):
                    w = plsc.load_gather(tb, [base[0] + dp])
                    acc_lo = coef[0] * plsc.bitcast(w << 16, jnp.float32)
                    acc_hi = coef[0] * plsc.bitcast(w & himask, jnp.float32)
                    for j in range(1, A):
                        w = plsc.load_gather(tb, [base[j] + dp])
                        acc_lo = acc_lo + coef[j] * plsc.bitcast(
                            w << 16, jnp.float32)
                        acc_hi = acc_hi + coef[j] * plsc.bitcast(
                            w & himask, jnp.float32)
                    plsc.store_scatter(
                        ob, [bloc, jnp.full((L,), fl * D + 2 * dp,
                                            jnp.int32)], acc_lo)
                    plsc.store_scatter(
                        ob, [bloc, jnp.full((L,), fl * D + 2 * dp + 1,
                                            jnp.int32)], acc_hi)

            pltpu.sync_copy(
                ob.at[:, pl.ds(0, NF * D)],
                y2_hbm.at[pl.ds(b0c, BC), pl.ds(f0 * D, NF * D)])
            return carry

        lax.fori_loop(0, NCH, cbody, 0)

    return k


def kernel(X, params):
    import numpy as np
    B, I, A = X.shape
    _, T, D = params.shape
    k = _make_kernel(B, I, A, T, D, BC=128)
    pw = jax.lax.bitcast_convert_type(
        params.astype(jnp.bfloat16).reshape(I * T * D // 2, 2),
        np.int32).reshape(-1)
    y2 = k(X.reshape(B, I * A), pw)
    return y2.reshape(B, I, D)


# speedup vs baseline: 1.8285x; 1.8285x over previous
import functools

import jax
import jax.numpy as jnp
from jax import lax
from jax.experimental import pallas as pl
from jax.experimental.pallas import tpu as pltpu
from jax.experimental.pallas import tpu_sc as plsc

L = 16
NC = 2
NS = 16
NW = NC * NS

_CES = [(0, 1), (2, 3), (4, 5), (6, 7),
        (0, 2), (1, 3), (4, 6), (5, 7),
        (1, 2), (5, 6),
        (0, 4), (1, 5), (2, 6), (3, 7),
        (2, 4), (3, 5),
        (1, 2), (3, 4), (5, 6)]


def _make_kernel(B, I, A, T, D, BC):
    NF = I // NW
    NG = BC // L
    NCH = B // BC
    HD = D // 2
    TS = HD + 1
    XS = NF * A + 1
    OS = NF * D + 1
    NTC = 2
    mesh = plsc.VectorSubcoreMesh(core_axis_name="c", subcore_axis_name="s",
                                  num_cores=NC, num_subcores=NS)

    @functools.partial(
        pl.kernel,
        out_type=jax.ShapeDtypeStruct((B, I * D), jnp.float32),
        mesh=mesh,
        scratch_types=[
            pltpu.VMEM((NTC * T * D,), jnp.float32),
            pltpu.VMEM((NF * T * TS,), jnp.int32),
            pltpu.VMEM((BC, XS), jnp.float32),
            pltpu.VMEM((BC, OS), jnp.float32),
        ],
        compiler_params=pltpu.CompilerParams(
            needs_layout_passes=False, use_tc_tiling_on_sc=False),
    )
    def k(x2_hbm, pflat_hbm, y2_hbm, traw, tb, xb, ob):
        wid = lax.axis_index("s") * NC + lax.axis_index("c")
        f0 = wid * NF
        lane = jnp.arange(L, dtype=jnp.int32)
        pay = [jnp.full((L,), 1 << j, jnp.int32) for j in range(A)]
        j0_off = (255 ^ 15) * TS

        rowsel = lane >> 3
        within = lane & 7
        for tc in range(NF // NTC):
            pltpu.sync_copy(
                pflat_hbm.at[pl.ds((f0 + tc * NTC) * (T * D), NTC * T * D)],
                traw)

            @plsc.parallel_loop(0, NTC * T // 2, 1, unroll=2)
            def _srow(r2):
                idx_e = r2 * (2 * D) + rowsel * D + within * 2
                ve = plsc.bitcast(plsc.load_gather(traw, [idx_e]), jnp.int32)
                vo = plsc.bitcast(plsc.load_gather(traw, [idx_e + 1]),
                                  jnp.int32)
                be = lax.shift_right_logical(
                    ve + 32767 + (lax.shift_right_logical(ve, 16) & 1), 16)
                bo = lax.shift_right_logical(
                    vo + 32767 + (lax.shift_right_logical(vo, 16) & 1), 16)
                w = (bo << 16) | be
                ri = r2 * 2 + rowsel
                m = ri & (T - 1)
                fc = ri >> 8
                s = m ^ (m >> 4)
                dst = ((tc * NTC + fc) * T + s) * TS + within
                plsc.store_scatter(tb, [dst], w)

        def cbody(ci, carry):
            b0c = ci * BC
            pltpu.sync_copy(
                x2_hbm.at[pl.ds(b0c, BC), pl.ds(f0 * A, NF * A)],
                xb.at[:, pl.ds(0, NF * A)])

            @plsc.parallel_loop(0, NF * NG, 1, unroll=2)
            def _group(t):
                fl = t // NG
                g = t - fl * NG
                tbase = fl * (T * TS)
                bloc = g * L + lane
                xrow = bloc * XS + fl * A
                v = [plsc.load_gather(xb, [bloc, jnp.full(
                        (L,), fl * A + j, jnp.int32)]) for j in range(A)]
                p = list(pay)
                for a, b in _CES:
                    c = v[a] <= v[b]
                    va, vb = v[a], v[b]
                    v[a] = jnp.where(c, va, vb)
                    v[b] = jnp.where(c, vb, va)
                    pa, pb = p[a], p[b]
                    p[a] = jnp.where(c, pa, pb)
                    p[b] = jnp.where(c, pb, pa)
                coef = [v[0]] + [v[j] - v[j - 1] for j in range(1, A)]
                m = [None] * A
                m[A - 1] = p[A - 1]
                for j in range(A - 2, 0, -1):
                    m[j] = m[j + 1] + p[j]
                base = [None] * A
                base[0] = jnp.full((L,), 0, jnp.int32) + (tbase + j0_off)
                for j in range(1, A):
                    s = m[j] ^ (m[j] >> 4)
                    base[j] = tbase + s * TS
                himask = jnp.full((L,), -65536, jnp.int32)
                for dp in range(HD):
                    w = plsc.load_gather(tb, [base[0] + dp])
                    acc_lo = coef[0] * plsc.bitcast(w << 16, jnp.float32)
                    acc_hi = coef[0] * plsc.bitcast(w & himask, jnp.float32)
                    for j in range(1, A):
                        w = plsc.load_gather(tb, [base[j] + dp])
                        acc_lo = acc_lo + coef[j] * plsc.bitcast(
                            w << 16, jnp.float32)
                        acc_hi = acc_hi + coef[j] * plsc.bitcast(
                            w & himask, jnp.float32)
                    plsc.store_scatter(
                        ob, [bloc, jnp.full((L,), fl * D + 2 * dp,
                                            jnp.int32)], acc_lo)
                    plsc.store_scatter(
                        ob, [bloc, jnp.full((L,), fl * D + 2 * dp + 1,
                                            jnp.int32)], acc_hi)

            pltpu.sync_copy(
                ob.at[:, pl.ds(0, NF * D)],
                y2_hbm.at[pl.ds(b0c, BC), pl.ds(f0 * D, NF * D)])
            return carry

        lax.fori_loop(0, NCH, cbody, 0)

    return k


def kernel(X, params):
    B, I, A = X.shape
    _, T, D = params.shape
    k = _make_kernel(B, I, A, T, D, BC=128)
    y2 = k(X.reshape(B, I * A), params.reshape(-1))
    return y2.reshape(B, I, D)


# speedup vs baseline: 1.9931x; 1.0901x over previous
import functools

import jax
import jax.numpy as jnp
from jax import lax
from jax.experimental import pallas as pl
from jax.experimental.pallas import tpu as pltpu
from jax.experimental.pallas import tpu_sc as plsc

L = 16
NC = 2
NS = 16
NW = NC * NS

_CES = [(0, 1), (2, 3), (4, 5), (6, 7),
        (0, 2), (1, 3), (4, 6), (5, 7),
        (1, 2), (5, 6),
        (0, 4), (1, 5), (2, 6), (3, 7),
        (2, 4), (3, 5),
        (1, 2), (3, 4), (5, 6)]


def _make_kernel(B, I, A, T, D, BC):
    NF = I // NW
    NG = BC // L
    NCH = B // BC
    HD = D // 2
    TS = HD + 1
    NTC = 2
    NIB = I // 128
    NDB = D // 8
    mesh = plsc.VectorSubcoreMesh(core_axis_name="c", subcore_axis_name="s",
                                  num_cores=NC, num_subcores=NS)

    @functools.partial(
        pl.kernel,
        out_type=jax.ShapeDtypeStruct((B, NDB, NIB, 8, 128), jnp.float32),
        mesh=mesh,
        scratch_types=[
            pltpu.VMEM((NTC * T * D,), jnp.float32),
            pltpu.VMEM((NF * T * TS,), jnp.int32),
            pltpu.VMEM((BC, 9, 9), jnp.float32),
            pltpu.VMEM((BC, 9, 9), jnp.float32),
            pltpu.VMEM((BC, 9, 9), jnp.float32),
        ],
        compiler_params=pltpu.CompilerParams(
            needs_layout_passes=False, use_tc_tiling_on_sc=False),
    )
    def k(x4_hbm, pflat_hbm, y5_hbm, traw, tb, xb, ob0, ob1):
        wid = lax.axis_index("s") * NC + lax.axis_index("c")
        f0 = wid * NF
        ib = wid // (128 // NF)
        ilo = (wid % (128 // NF)) * NF
        lane = jnp.arange(L, dtype=jnp.int32)
        pay = [jnp.full((L,), 1 << j, jnp.int32) for j in range(A)]
        j0_off = (255 ^ 15) * TS

        rowsel = lane >> 3
        within = lane & 7
        for tc in range(NF // NTC):
            pltpu.sync_copy(
                pflat_hbm.at[pl.ds((f0 + tc * NTC) * (T * D), NTC * T * D)],
                traw)

            @plsc.parallel_loop(0, NTC * T // 2, 1, unroll=2)
            def _srow(r2):
                idx_e = r2 * (2 * D) + rowsel * D + within * 2
                ve = plsc.bitcast(plsc.load_gather(traw, [idx_e]), jnp.int32)
                vo = plsc.bitcast(plsc.load_gather(traw, [idx_e + 1]),
                                  jnp.int32)
                be = lax.shift_right_logical(
                    ve + 32767 + (lax.shift_right_logical(ve, 16) & 1), 16)
                bo = lax.shift_right_logical(
                    vo + 32767 + (lax.shift_right_logical(vo, 16) & 1), 16)
                w = (bo << 16) | be
                ri = r2 * 2 + rowsel
                m = ri & (T - 1)
                fc = ri >> 8
                s = m ^ (m >> 4)
                dst = ((tc * NTC + fc) * T + s) * TS + within
                plsc.store_scatter(tb, [dst], w)

        def cbody(ci, carry):
            b0c = ci * BC
            pltpu.sync_copy(
                x4_hbm.at[pl.ds(b0c, BC), ib, :, pl.ds(ilo, NF)],
                xb.at[:, pl.ds(0, A), pl.ds(0, NF)])

            @plsc.parallel_loop(0, NF * NG, 1, unroll=2)
            def _group(t):
                fl = t // NG
                g = t - fl * NG
                tbase = fl * (T * TS)
                bloc = g * L + lane
                fls = jnp.full((L,), fl, jnp.int32)
                v = [plsc.load_gather(
                        xb, [bloc, jnp.full((L,), j, jnp.int32), fls])
                     for j in range(A)]
                p = list(pay)
                for a, b in _CES:
                    c = v[a] <= v[b]
                    va, vb = v[a], v[b]
                    v[a] = jnp.where(c, va, vb)
                    v[b] = jnp.where(c, vb, va)
                    pa, pb = p[a], p[b]
                    p[a] = jnp.where(c, pa, pb)
                    p[b] = jnp.where(c, pb, pa)
                coef = [v[0]] + [v[j] - v[j - 1] for j in range(1, A)]
                m = [None] * A
                m[A - 1] = p[A - 1]
                for j in range(A - 2, 0, -1):
                    m[j] = m[j + 1] + p[j]
                base = [None] * A
                base[0] = jnp.full((L,), 0, jnp.int32) + (tbase + j0_off)
                for j in range(1, A):
                    s = m[j] ^ (m[j] >> 4)
                    base[j] = tbase + s * TS
                himask = jnp.full((L,), -65536, jnp.int32)
                obs = [ob0, ob1]
                for dp in range(HD):
                    w = plsc.load_gather(tb, [base[0] + dp])
                    acc_lo = coef[0] * plsc.bitcast(w << 16, jnp.float32)
                    acc_hi = coef[0] * plsc.bitcast(w & himask, jnp.float32)
                    for j in range(1, A):
                        w = plsc.load_gather(tb, [base[j] + dp])
                        acc_lo = acc_lo + coef[j] * plsc.bitcast(
                            w << 16, jnp.float32)
                        acc_hi = acc_hi + coef[j] * plsc.bitcast(
                            w & himask, jnp.float32)
                    od = obs[dp >> 2]
                    dm_lo = (2 * dp) & 7
                    plsc.store_scatter(
                        od, [bloc, jnp.full((L,), dm_lo, jnp.int32), fls],
                        acc_lo)
                    plsc.store_scatter(
                        od, [bloc, jnp.full((L,), dm_lo + 1, jnp.int32), fls],
                        acc_hi)

            for db in range(NDB):
                ob = [ob0, ob1][db]
                pltpu.sync_copy(
                    ob.at[:, pl.ds(0, 8), pl.ds(0, NF)],
                    y5_hbm.at[pl.ds(b0c, BC), db, ib, :, pl.ds(ilo, NF)])
            return carry

        lax.fori_loop(0, NCH, cbody, 0)

    return k


def kernel(X, params):
    B, I, A = X.shape
    _, T, D = params.shape
    NIB = I // 128
    NDB = D // 8
    k = _make_kernel(B, I, A, T, D, BC=128)
    x4 = X.transpose(0, 2, 1).reshape(B, A, NIB, 128).transpose(0, 2, 1, 3)
    y5 = k(x4, params.reshape(-1))
    return (y5.transpose(0, 2, 4, 1, 3)
            .reshape(B, I, D))


# speedup vs baseline: 2.2559x; 1.1318x over previous
import functools

import jax
import jax.numpy as jnp
from jax import lax
from jax.experimental import pallas as pl
from jax.experimental.pallas import tpu as pltpu
from jax.experimental.pallas import tpu_sc as plsc

L = 16
NC = 2
NS = 16
NW = NC * NS

_CES = [(0, 1), (2, 3), (4, 5), (6, 7),
        (0, 2), (1, 3), (4, 6), (5, 7),
        (1, 2), (5, 6),
        (0, 4), (1, 5), (2, 6), (3, 7),
        (2, 4), (3, 5),
        (1, 2), (3, 4), (5, 6)]


def _make_kernel(B, I, A, T, D, BC):
    NF = I // NW
    NG = BC // L
    NCH = B // BC
    HD = D // 2
    TS = HD + 1
    XS = NF * A + 1
    OS = NF * D + 1
    NTC = 2
    NIB = I // 128
    NDB = D // 8
    mesh = plsc.VectorSubcoreMesh(core_axis_name="c", subcore_axis_name="s",
                                  num_cores=NC, num_subcores=NS)

    @functools.partial(
        pl.kernel,
        out_type=jax.ShapeDtypeStruct((B, NDB, NIB, 8, 128), jnp.float32),
        mesh=mesh,
        scratch_types=[
            pltpu.VMEM((NTC * T * D,), jnp.float32),
            pltpu.VMEM((NF * T * TS,), jnp.int32),
            [pltpu.VMEM((BC, A, NF), jnp.float32)
             for _ in range(2)],
            pltpu.VMEM((BC * XS,), jnp.float32),
            pltpu.VMEM((BC * OS,), jnp.float32),
            [pltpu.VMEM((BC, NDB, 8, NF), jnp.float32)
             for _ in range(2)],
            [pltpu.SemaphoreType.DMA for _ in range(2)],
            [pltpu.SemaphoreType.DMA for _ in range(2)],
        ],
        compiler_params=pltpu.CompilerParams(
            needs_layout_passes=False, use_tc_tiling_on_sc=False),
    )
    def k(x4_hbm, pflat_hbm, y5_hbm, traw, tb, stx, xf, of, sty, sxs, sys):
        wid = lax.axis_index("s") * NC + lax.axis_index("c")
        f0 = wid * NF
        ib = wid // (128 // NF)
        ilo = (wid % (128 // NF)) * NF
        lane = jnp.arange(L, dtype=jnp.int32)
        rowsel = lane >> 3
        within = lane & 7
        pay = [jnp.full((L,), 1 << j, jnp.int32) for j in range(A)]
        j0_off = (255 ^ 15) * TS

        def start_x(ci):
            return pltpu.async_copy(
                x4_hbm.at[pl.ds(ci * BC, BC), ib, :, pl.ds(ilo, NF)],
                stx[ci % 2], sxs[ci % 2])

        x_descs = [None, None]
        x_descs[0] = start_x(0)

        for tc in range(NF // NTC):
            pltpu.sync_copy(
                pflat_hbm.at[pl.ds((f0 + tc * NTC) * (T * D), NTC * T * D)],
                traw)

            @plsc.parallel_loop(0, NTC * T // 2, 1, unroll=2)
            def _srow(r2):
                idx_e = r2 * (2 * D) + rowsel * D + within * 2
                ve = plsc.bitcast(plsc.load_gather(traw, [idx_e]), jnp.int32)
                vo = plsc.bitcast(plsc.load_gather(traw, [idx_e + 1]),
                                  jnp.int32)
                be = lax.shift_right_logical(
                    ve + 32767 + (lax.shift_right_logical(ve, 16) & 1), 16)
                bo = lax.shift_right_logical(
                    vo + 32767 + (lax.shift_right_logical(vo, 16) & 1), 16)
                w = (bo << 16) | be
                ri = r2 * 2 + rowsel
                m = ri & (T - 1)
                fc = ri >> 8
                s = m ^ (m >> 4)
                dst = ((tc * NTC + fc) * T + s) * TS + within
                plsc.store_scatter(tb, [dst], w)

        out_descs = [None, None]
        for ci in range(NCH):
            b0c = ci * BC
            par = ci % 2
            x_descs[par].wait()
            if ci + 1 < NCH:
                x_descs[1 - par] = start_x(ci + 1)

            stxc = stx[par]

            @plsc.parallel_loop(0, BC * (NF * A // L), 1, unroll=4)
            def _xrep(r):
                b = r >> 2
                kk = (r & 3) * L
                v = plsc.load_gather(
                    stxc, [jnp.full((L,), b, jnp.int32),
                           (kk + lane) >> 3, within])
                plsc.store_scatter(xf, [b * XS + kk + lane], v)

            @plsc.parallel_loop(0, NF * NG, 1, unroll=2)
            def _group(t):
                fl = t // NG
                g = t - fl * NG
                tbase = fl * (T * TS)
                bloc = g * L + lane
                ox = bloc * XS + fl
                v = [plsc.load_gather(xf, [ox + j * NF]) for j in range(A)]
                p = list(pay)
                for a, b in _CES:
                    c = v[a] <= v[b]
                    va, vb = v[a], v[b]
                    v[a] = jnp.where(c, va, vb)
                    v[b] = jnp.where(c, vb, va)
                    pa, pb = p[a], p[b]
                    p[a] = jnp.where(c, pa, pb)
                    p[b] = jnp.where(c, pb, pa)
                coef = [v[0]] + [v[j] - v[j - 1] for j in range(1, A)]
                m = [None] * A
                m[A - 1] = p[A - 1]
                for j in range(A - 2, 0, -1):
                    m[j] = m[j + 1] + p[j]
                base = [None] * A
                base[0] = jnp.full((L,), 0, jnp.int32) + (tbase + j0_off)
                for j in range(1, A):
                    s = m[j] ^ (m[j] >> 4)
                    base[j] = tbase + s * TS
                himask = jnp.full((L,), -65536, jnp.int32)
                oo = bloc * OS + fl
                for dp in range(HD):
                    w = plsc.load_gather(tb, [base[0] + dp])
                    acc_lo = coef[0] * plsc.bitcast(w << 16, jnp.float32)
                    acc_hi = coef[0] * plsc.bitcast(w & himask, jnp.float32)
                    for j in range(1, A):
                        w = plsc.load_gather(tb, [base[j] + dp])
                        acc_lo = acc_lo + coef[j] * plsc.bitcast(
                            w << 16, jnp.float32)
                        acc_hi = acc_hi + coef[j] * plsc.bitcast(
                            w & himask, jnp.float32)
                    d_lo, d_hi = 2 * dp, 2 * dp + 1
                    plsc.store_scatter(
                        of, [oo + ((d_lo >> 3) * 64 + (d_lo & 7) * 8)],
                        acc_lo)
                    plsc.store_scatter(
                        of, [oo + ((d_hi >> 3) * 64 + (d_hi & 7) * 8)],
                        acc_hi)

            if out_descs[par] is not None:
                out_descs[par].wait()
            styc = sty[par]

            @plsc.parallel_loop(0, BC * (NF * D // L), 1, unroll=4)
            def _yrep(r):
                b = r >> 3
                k = r & 7
                kk = k * L
                v = plsc.load_gather(of, [b * OS + kk + lane])
                plsc.store_scatter(
                    styc, [jnp.full((L,), b, jnp.int32),
                           jnp.full((L,), k >> 2, jnp.int32),
                           ((kk + lane) >> 3) & 7, within], v)

            out_descs[par] = pltpu.async_copy(
                styc, y5_hbm.at[pl.ds(b0c, BC), :, ib, :, pl.ds(ilo, NF)],
                sys[par])

        for par in range(2):
            if out_descs[par] is not None:
                out_descs[par].wait()

    return k


def kernel(X, params):
    B, I, A = X.shape
    _, T, D = params.shape
    NIB = I // 128
    k = _make_kernel(B, I, A, T, D, BC=128)
    x4 = X.transpose(0, 2, 1).reshape(B, A, NIB, 128).transpose(0, 2, 1, 3)
    y5 = k(x4, params.reshape(-1))
    return y5.transpose(0, 2, 4, 1, 3).reshape(B, I, D)


# speedup vs baseline: 2.2627x; 1.0030x over previous
import functools

import jax
import jax.numpy as jnp
from jax import lax
from jax.experimental import pallas as pl
from jax.experimental.pallas import tpu as pltpu
from jax.experimental.pallas import tpu_sc as plsc

L = 16
NC = 2
NS = 16
NW = NC * NS

_CES = [(0, 1), (2, 3), (4, 5), (6, 7),
        (0, 2), (1, 3), (4, 6), (5, 7),
        (1, 2), (5, 6),
        (0, 4), (1, 5), (2, 6), (3, 7),
        (2, 4), (3, 5),
        (1, 2), (3, 4), (5, 6)]


def _make_kernel(B, I, A, T, D, BC):
    NF = I // NW
    NG = BC // L
    NCH = B // BC
    HD = D // 2
    TS = HD + 1
    XS = NF * A + 1
    OS = NF * D + 1
    NTC = 2
    NIB = I // 128
    NDB = D // 8
    mesh = plsc.VectorSubcoreMesh(core_axis_name="c", subcore_axis_name="s",
                                  num_cores=NC, num_subcores=NS)

    @functools.partial(
        pl.kernel,
        out_type=jax.ShapeDtypeStruct((B, NDB, NIB, 8, 128), jnp.float32),
        mesh=mesh,
        scratch_types=[
            pltpu.VMEM((NTC * T * D,), jnp.float32),
            pltpu.VMEM((NF * T * TS,), jnp.int32),
            [pltpu.VMEM((BC, A, NF), jnp.float32)
             for _ in range(2)],
            pltpu.VMEM((BC * XS,), jnp.float32),
            pltpu.VMEM((BC * OS,), jnp.float32),
            [pltpu.VMEM((BC, NDB, 8, NF), jnp.float32)
             for _ in range(2)],
            [pltpu.SemaphoreType.DMA for _ in range(2)],
            [pltpu.SemaphoreType.DMA for _ in range(2)],
        ],
        compiler_params=pltpu.CompilerParams(
            needs_layout_passes=False, use_tc_tiling_on_sc=False),
    )
    def k(x4_hbm, pflat_hbm, y5_hbm, traw, tb, stx, xf, of, sty, sxs, sys):
        wid = lax.axis_index("s") * NC + lax.axis_index("c")
        f0 = wid * NF
        ib = wid // (128 // NF)
        ilo = (wid % (128 // NF)) * NF
        lane = jnp.arange(L, dtype=jnp.int32)
        rowsel = lane >> 3
        within = lane & 7
        pay = [jnp.full((L,), 1 << j, jnp.int32) for j in range(A)]
        j0_off = (255 ^ 15) * TS

        def start_x(ci):
            return pltpu.async_copy(
                x4_hbm.at[pl.ds(ci * BC, BC), ib, :, pl.ds(ilo, NF)],
                stx[ci % 2], sxs[ci % 2])

        x_descs = [None, None]
        x_descs[0] = start_x(0)

        for tc in range(NF // NTC):
            pltpu.sync_copy(
                pflat_hbm.at[pl.ds((f0 + tc * NTC) * (T * D), NTC * T * D)],
                traw)

            @plsc.parallel_loop(0, NTC * T // 2, 1, unroll=2)
            def _srow(r2):
                idx_e = r2 * (2 * D) + rowsel * D + within * 2
                ve = plsc.bitcast(plsc.load_gather(traw, [idx_e]), jnp.int32)
                vo = plsc.bitcast(plsc.load_gather(traw, [idx_e + 1]),
                                  jnp.int32)
                be = lax.shift_right_logical(
                    ve + 32767 + (lax.shift_right_logical(ve, 16) & 1), 16)
                bo = lax.shift_right_logical(
                    vo + 32767 + (lax.shift_right_logical(vo, 16) & 1), 16)
                w = (bo << 16) | be
                ri = r2 * 2 + rowsel
                m = ri & (T - 1)
                fc = ri >> 8
                s = m ^ (m >> 4)
                dst = ((tc * NTC + fc) * T + s) * TS + within
                plsc.store_scatter(tb, [dst], w)

        out_descs = [None, None]
        for ci in range(NCH):
            b0c = ci * BC
            par = ci % 2
            x_descs[par].wait()
            if ci + 1 < NCH:
                x_descs[1 - par] = start_x(ci + 1)

            stxc = stx[par]

            @plsc.parallel_loop(0, BC * (NF * A // L), 1, unroll=4)
            def _xrep(r):
                b = r >> 2
                kk = (r & 3) * L
                v = plsc.load_gather(
                    stxc, [jnp.full((L,), b, jnp.int32),
                           (kk + lane) >> 3, within])
                plsc.store_scatter(xf, [b * XS + kk + lane], v)

            @plsc.parallel_loop(0, NF * NG, 1, unroll=4)
            def _group(t):
                fl = t // NG
                g = t - fl * NG
                tbase = fl * (T * TS)
                bloc = g * L + lane
                ox = bloc * XS + fl
                v = [plsc.load_gather(xf, [ox + j * NF]) for j in range(A)]
                p = list(pay)
                for a, b in _CES:
                    c = v[a] <= v[b]
                    va, vb = v[a], v[b]
                    v[a] = jnp.where(c, va, vb)
                    v[b] = jnp.where(c, vb, va)
                    pa, pb = p[a], p[b]
                    p[a] = jnp.where(c, pa, pb)
                    p[b] = jnp.where(c, pb, pa)
                coef = [v[0]] + [v[j] - v[j - 1] for j in range(1, A)]
                m = [None] * A
                m[A - 1] = p[A - 1]
                for j in range(A - 2, 0, -1):
                    m[j] = m[j + 1] + p[j]
                base = [None] * A
                base[0] = jnp.full((L,), 0, jnp.int32) + (tbase + j0_off)
                for j in range(1, A):
                    s = m[j] ^ (m[j] >> 4)
                    base[j] = tbase + s * TS
                himask = jnp.full((L,), -65536, jnp.int32)
                oo = bloc * OS + fl
                for dp in range(HD):
                    w = plsc.load_gather(tb, [base[0] + dp])
                    acc_lo = coef[0] * plsc.bitcast(w << 16, jnp.float32)
                    acc_hi = coef[0] * plsc.bitcast(w & himask, jnp.float32)
                    for j in range(1, A):
                        w = plsc.load_gather(tb, [base[j] + dp])
                        acc_lo = acc_lo + coef[j] * plsc.bitcast(
                            w << 16, jnp.float32)
                        acc_hi = acc_hi + coef[j] * plsc.bitcast(
                            w & himask, jnp.float32)
                    d_lo, d_hi = 2 * dp, 2 * dp + 1
                    plsc.store_scatter(
                        of, [oo + ((d_lo >> 3) * 64 + (d_lo & 7) * 8)],
                        acc_lo)
                    plsc.store_scatter(
                        of, [oo + ((d_hi >> 3) * 64 + (d_hi & 7) * 8)],
                        acc_hi)

            if out_descs[par] is not None:
                out_descs[par].wait()
            styc = sty[par]

            @plsc.parallel_loop(0, BC * (NF * D // L), 1, unroll=4)
            def _yrep(r):
                b = r >> 3
                k = r & 7
                kk = k * L
                v = plsc.load_gather(of, [b * OS + kk + lane])
                plsc.store_scatter(
                    styc, [jnp.full((L,), b, jnp.int32),
                           jnp.full((L,), k >> 2, jnp.int32),
                           ((kk + lane) >> 3) & 7, within], v)

            out_descs[par] = pltpu.async_copy(
                styc, y5_hbm.at[pl.ds(b0c, BC), :, ib, :, pl.ds(ilo, NF)],
                sys[par])

        for par in range(2):
            if out_descs[par] is not None:
                out_descs[par].wait()

    return k


def kernel(X, params):
    B, I, A = X.shape
    _, T, D = params.shape
    NIB = I // 128
    k = _make_kernel(B, I, A, T, D, BC=128)
    x4 = X.transpose(0, 2, 1).reshape(B, A, NIB, 128).transpose(0, 2, 1, 3)
    y5 = k(x4, params.reshape(-1))
    return y5.transpose(0, 2, 4, 1, 3).reshape(B, I, D)


# speedup vs baseline: 2.3756x; 1.0499x over previous
import functools

import jax
import jax.numpy as jnp
from jax import lax
from jax.experimental import pallas as pl
from jax.experimental.pallas import tpu as pltpu
from jax.experimental.pallas import tpu_sc as plsc

L = 16
NC = 2
NS = 16
NW = NC * NS

_CES = [(0, 1), (2, 3), (4, 5), (6, 7),
        (0, 2), (1, 3), (4, 6), (5, 7),
        (1, 2), (5, 6),
        (0, 4), (1, 5), (2, 6), (3, 7),
        (2, 4), (3, 5),
        (1, 2), (3, 4), (5, 6)]


def _make_kernel(B, I, A, T, D, BC):
    NF = I // NW
    NG = BC // L
    NCH = B // BC
    HD = D // 2
    TS = HD + 1
    XS = NF * A + 1
    OS = NF * D + 1
    NTC = 2
    NIB = I // 128
    NDB = D // 8
    mesh = plsc.VectorSubcoreMesh(core_axis_name="c", subcore_axis_name="s",
                                  num_cores=NC, num_subcores=NS)

    @functools.partial(
        pl.kernel,
        out_type=jax.ShapeDtypeStruct((B, NDB, NIB, 8, 128), jnp.float32),
        mesh=mesh,
        scratch_types=[
            pltpu.VMEM((NTC * T * D,), jnp.float32),
            pltpu.VMEM((NF * T * TS,), jnp.int32),
            [pltpu.VMEM((BC, A, NF), jnp.float32)
             for _ in range(2)],
            pltpu.VMEM((BC * XS,), jnp.float32),
            pltpu.VMEM((BC * OS,), jnp.float32),
            [pltpu.VMEM((BC, NDB, 8, NF), jnp.float32)
             for _ in range(2)],
            [pltpu.SemaphoreType.DMA for _ in range(2)],
            [pltpu.SemaphoreType.DMA for _ in range(2)],
        ],
        compiler_params=pltpu.CompilerParams(
            needs_layout_passes=False, use_tc_tiling_on_sc=False),
    )
    def k(x4_hbm, pflat_hbm, y5_hbm, traw, tb, stx, xf, of, sty, sxs, sys):
        wid = lax.axis_index("s") * NC + lax.axis_index("c")
        f0 = wid * NF
        ib = wid // (128 // NF)
        ilo = (wid % (128 // NF)) * NF
        lane = jnp.arange(L, dtype=jnp.int32)
        rowsel = lane >> 3
        within = lane & 7
        pay = [jnp.full((L,), 1 << j, jnp.int32) for j in range(A)]
        j0_off = (255 ^ 15) * TS

        def start_x(ci):
            return pltpu.async_copy(
                x4_hbm.at[pl.ds(ci * BC, BC), ib, :, pl.ds(ilo, NF)],
                stx[ci % 2], sxs[ci % 2])

        x_descs = [None, None]
        x_descs[0] = start_x(0)

        for tc in range(NF // NTC):
            pltpu.sync_copy(
                pflat_hbm.at[pl.ds((f0 + tc * NTC) * (T * D), NTC * T * D)],
                traw)

            @plsc.parallel_loop(0, NTC * T // 2, 1, unroll=2)
            def _srow(r2):
                idx_e = r2 * (2 * D) + rowsel * D + within * 2
                ve = plsc.bitcast(plsc.load_gather(traw, [idx_e]), jnp.int32)
                vo = plsc.bitcast(plsc.load_gather(traw, [idx_e + 1]),
                                  jnp.int32)
                be = lax.shift_right_logical(
                    ve + 32767 + (lax.shift_right_logical(ve, 16) & 1), 16)
                bo = lax.shift_right_logical(
                    vo + 32767 + (lax.shift_right_logical(vo, 16) & 1), 16)
                w = (bo << 16) | be
                ri = r2 * 2 + rowsel
                m = ri & (T - 1)
                fc = ri >> 8
                s = m ^ (m >> 4)
                dst = ((tc * NTC + fc) * T + s) * TS + within
                plsc.store_scatter(tb, [dst], w)

        out_descs = [None, None]
        for ci in range(NCH):
            b0c = ci * BC
            par = ci % 2
            x_descs[par].wait()
            if ci + 1 < NCH:
                x_descs[1 - par] = start_x(ci + 1)

            stxc = stx[par]

            @plsc.parallel_loop(0, BC * (NF * A // L), 1, unroll=4)
            def _xrep(r):
                b = r >> 2
                kk = (r & 3) * L
                v = plsc.load_gather(
                    stxc, [jnp.full((L,), b, jnp.int32),
                           (kk + lane) >> 3, within])
                plsc.store_scatter(xf, [b * XS + kk + lane], v)

            @plsc.parallel_loop(0, NF * NG, 1, unroll=4)
            def _group(t):
                fl = t // NG
                g = t - fl * NG
                tbase = fl * (T * TS)
                bloc = g * L + lane
                ox = bloc * XS + fl
                v = [plsc.load_gather(xf, [ox + j * NF]) for j in range(A)]
                p = list(pay)
                for a, b in _CES:
                    c = v[a] <= v[b]
                    va, vb = v[a], v[b]
                    v[a] = jnp.where(c, va, vb)
                    v[b] = jnp.where(c, vb, va)
                    pa, pb = p[a], p[b]
                    p[a] = jnp.where(c, pa, pb)
                    p[b] = jnp.where(c, pb, pa)
                coef = [v[0]] + [v[j] - v[j - 1] for j in range(1, A)]
                m = [None] * A
                m[A - 1] = p[A - 1]
                for j in range(A - 2, 0, -1):
                    m[j] = m[j + 1] + p[j]
                base = [None] * A
                base[0] = jnp.full((L,), 0, jnp.int32) + (tbase + j0_off)
                for j in range(1, A):
                    s = m[j] ^ (m[j] >> 4)
                    base[j] = tbase + s * TS
                himask = jnp.full((L,), -65536, jnp.int32)
                oo = bloc * OS + fl
                acc_lo = [None] * HD
                acc_hi = [None] * HD
                for j in range(A):
                    for dp in range(HD):
                        w = plsc.load_gather(tb, [base[j] + dp])
                        plo = coef[j] * plsc.bitcast(w << 16, jnp.float32)
                        phi = coef[j] * plsc.bitcast(w & himask, jnp.float32)
                        if j == 0:
                            acc_lo[dp] = plo
                            acc_hi[dp] = phi
                        else:
                            acc_lo[dp] = acc_lo[dp] + plo
                            acc_hi[dp] = acc_hi[dp] + phi
                for dp in range(HD):
                    d_lo, d_hi = 2 * dp, 2 * dp + 1
                    plsc.store_scatter(
                        of, [oo + ((d_lo >> 3) * 64 + (d_lo & 7) * 8)],
                        acc_lo[dp])
                    plsc.store_scatter(
                        of, [oo + ((d_hi >> 3) * 64 + (d_hi & 7) * 8)],
                        acc_hi[dp])

            if out_descs[par] is not None:
                out_descs[par].wait()
            styc = sty[par]

            @plsc.parallel_loop(0, BC * (NF * D // L), 1, unroll=4)
            def _yrep(r):
                b = r >> 3
                k = r & 7
                kk = k * L
                v = plsc.load_gather(of, [b * OS + kk + lane])
                plsc.store_scatter(
                    styc, [jnp.full((L,), b, jnp.int32),
                           jnp.full((L,), k >> 2, jnp.int32),
                           ((kk + lane) >> 3) & 7, within], v)

            out_descs[par] = pltpu.async_copy(
                styc, y5_hbm.at[pl.ds(b0c, BC), :, ib, :, pl.ds(ilo, NF)],
                sys[par])

        for par in range(2):
            if out_descs[par] is not None:
                out_descs[par].wait()

    return k


def kernel(X, params):
    B, I, A = X.shape
    _, T, D = params.shape
    NIB = I // 128
    k = _make_kernel(B, I, A, T, D, BC=128)
    x4 = X.transpose(0, 2, 1).reshape(B, A, NIB, 128).transpose(0, 2, 1, 3)
    y5 = k(x4, params.reshape(-1))
    return y5.transpose(0, 2, 4, 1, 3).reshape(B, I, D)


# speedup vs baseline: 2.4616x; 1.0362x over previous
import functools

import jax
import jax.numpy as jnp
from jax import lax
from jax.experimental import pallas as pl
from jax.experimental.pallas import tpu as pltpu
from jax.experimental.pallas import tpu_sc as plsc

L = 16
NC = 2
NS = 16
NW = NC * NS

_CES = [(0, 1), (2, 3), (4, 5), (6, 7),
        (0, 2), (1, 3), (4, 6), (5, 7),
        (1, 2), (5, 6),
        (0, 4), (1, 5), (2, 6), (3, 7),
        (2, 4), (3, 5),
        (1, 2), (3, 4), (5, 6)]


def _make_kernel(B, I, A, T, D, BC):
    NF = I // NW
    NG = BC // L
    NCH = B // BC
    HD = D // 2
    TS = HD + 1
    XS = NF * A + 1
    OS = NF * D + 1
    NTC = 2
    NIB = I // 128
    NDB = D // 8
    mesh = plsc.VectorSubcoreMesh(core_axis_name="c", subcore_axis_name="s",
                                  num_cores=NC, num_subcores=NS)

    @functools.partial(
        pl.kernel,
        out_type=jax.ShapeDtypeStruct((B, NDB, NIB, 8, 128), jnp.float32),
        mesh=mesh,
        scratch_types=[
            pltpu.VMEM((NTC * T * D,), jnp.float32),
            pltpu.VMEM((NF * T * TS,), jnp.int32),
            [pltpu.VMEM((BC, A, NF), jnp.float32)
             for _ in range(2)],
            pltpu.VMEM((BC * XS,), jnp.float32),
            pltpu.VMEM((BC * OS,), jnp.float32),
            [pltpu.VMEM((BC, NDB, 8, NF), jnp.float32)
             for _ in range(2)],
            [pltpu.SemaphoreType.DMA for _ in range(2)],
            [pltpu.SemaphoreType.DMA for _ in range(2)],
        ],
        compiler_params=pltpu.CompilerParams(
            needs_layout_passes=False, use_tc_tiling_on_sc=False),
    )
    def k(x4_hbm, pflat_hbm, y5_hbm, traw, tb, stx, xf, of, sty, sxs, sys):
        wid = lax.axis_index("s") * NC + lax.axis_index("c")
        f0 = wid * NF
        ib = wid // (128 // NF)
        ilo = (wid % (128 // NF)) * NF
        lane = jnp.arange(L, dtype=jnp.int32)
        rowsel = lane >> 3
        within = lane & 7
        pay = [jnp.full((L,), 1 << j, jnp.int32) for j in range(A)]
        j0_off = (255 ^ 15) * TS

        def start_x(ci):
            return pltpu.async_copy(
                x4_hbm.at[pl.ds(ci * BC, BC), ib, :, pl.ds(ilo, NF)],
                stx[ci % 2], sxs[ci % 2])

        x_descs = [None, None]
        x_descs[0] = start_x(0)

        for tc in range(NF // NTC):
            pltpu.sync_copy(
                pflat_hbm.at[pl.ds((f0 + tc * NTC) * (T * D), NTC * T * D)],
                traw)

            @plsc.parallel_loop(0, NTC * T // 2, 1, unroll=2)
            def _srow(r2):
                idx_e = r2 * (2 * D) + rowsel * D + within * 2
                ve = plsc.bitcast(plsc.load_gather(traw, [idx_e]), jnp.int32)
                vo = plsc.bitcast(plsc.load_gather(traw, [idx_e + 1]),
                                  jnp.int32)
                be = lax.shift_right_logical(
                    ve + 32767 + (lax.shift_right_logical(ve, 16) & 1), 16)
                bo = lax.shift_right_logical(
                    vo + 32767 + (lax.shift_right_logical(vo, 16) & 1), 16)
                w = (bo << 16) | be
                ri = r2 * 2 + rowsel
                m = ri & (T - 1)
                fc = ri >> 8
                s = m ^ (m >> 4)
                dst = ((tc * NTC + fc) * T + s) * TS + within
                plsc.store_scatter(tb, [dst], w)

        out_descs = [None, None]
        for ci in range(NCH):
            b0c = ci * BC
            par = ci % 2
            x_descs[par].wait()
            if ci + 1 < NCH:
                x_descs[1 - par] = start_x(ci + 1)

            stxc = stx[par]

            @plsc.parallel_loop(0, BC * (NF * A // L), 1, unroll=4)
            def _xrep(r):
                b = r >> 2
                kk = (r & 3) * L
                v = plsc.load_gather(
                    stxc, [jnp.full((L,), b, jnp.int32),
                           (kk + lane) >> 3, within])
                plsc.store_scatter(xf, [b * XS + kk + lane], v)

            @plsc.parallel_loop(0, NF * NG, 1, unroll=2)
            def _group(t):
                fl = t // NG
                g = t - fl * NG
                tbase = fl * (T * TS)
                bloc = g * L + lane
                ox = bloc * XS + fl
                v = [plsc.load_gather(xf, [ox + j * NF]) for j in range(A)]
                p = list(pay)
                for a, b in _CES:
                    c = v[a] <= v[b]
                    va, vb = v[a], v[b]
                    v[a] = jnp.where(c, va, vb)
                    v[b] = jnp.where(c, vb, va)
                    pa, pb = p[a], p[b]
                    p[a] = jnp.where(c, pa, pb)
                    p[b] = jnp.where(c, pb, pa)
                coef = [v[0]] + [v[j] - v[j - 1] for j in range(1, A)]
                m = [None] * A
                m[A - 1] = p[A - 1]
                for j in range(A - 2, 0, -1):
                    m[j] = m[j + 1] + p[j]
                base = [None] * A
                base[0] = jnp.full((L,), 0, jnp.int32) + (tbase + j0_off)
                for j in range(1, A):
                    s = m[j] ^ (m[j] >> 4)
                    base[j] = tbase + s * TS
                himask = jnp.full((L,), -65536, jnp.int32)
                oo = bloc * OS + fl
                acc_lo = [None] * HD
                acc_hi = [None] * HD
                for j in range(A):
                    for dp in range(HD):
                        w = plsc.load_gather(tb, [base[j] + dp])
                        plo = coef[j] * plsc.bitcast(w << 16, jnp.float32)
                        phi = coef[j] * plsc.bitcast(w & himask, jnp.float32)
                        if j == 0:
                            acc_lo[dp] = plo
                            acc_hi[dp] = phi
                        else:
                            acc_lo[dp] = acc_lo[dp] + plo
                            acc_hi[dp] = acc_hi[dp] + phi
                for dp in range(HD):
                    d_lo, d_hi = 2 * dp, 2 * dp + 1
                    plsc.store_scatter(
                        of, [oo + ((d_lo >> 3) * 64 + (d_lo & 7) * 8)],
                        acc_lo[dp])
                    plsc.store_scatter(
                        of, [oo + ((d_hi >> 3) * 64 + (d_hi & 7) * 8)],
                        acc_hi[dp])

            if out_descs[par] is not None:
                out_descs[par].wait()
            styc = sty[par]

            @plsc.parallel_loop(0, BC * (NF * D // L), 1, unroll=4)
            def _yrep(r):
                b = r >> 3
                k = r & 7
                kk = k * L
                v = plsc.load_gather(of, [b * OS + kk + lane])
                plsc.store_scatter(
                    styc, [jnp.full((L,), b, jnp.int32),
                           jnp.full((L,), k >> 2, jnp.int32),
                           ((kk + lane) >> 3) & 7, within], v)

            out_descs[par] = pltpu.async_copy(
                styc, y5_hbm.at[pl.ds(b0c, BC), :, ib, :, pl.ds(ilo, NF)],
                sys[par])

        for par in range(2):
            if out_descs[par] is not None:
                out_descs[par].wait()

    return k


def kernel(X, params):
    B, I, A = X.shape
    _, T, D = params.shape
    NIB = I // 128
    k = _make_kernel(B, I, A, T, D, BC=128)
    x4 = X.transpose(0, 2, 1).reshape(B, A, NIB, 128).transpose(0, 2, 1, 3)
    y5 = k(x4, params.reshape(-1))
    return y5.transpose(0, 2, 4, 1, 3).reshape(B, I, D)
